# baseline (speedup 1.0000x reference)
import functools

import jax
import jax.numpy as jnp
from jax import lax
from jax.experimental import pallas as pl
from jax.experimental.pallas import tpu as pltpu
from jax.experimental.pallas import tpu_sc as plsc

_EMBED = 32
_HALF = 256


def _dot_kernel(uid_hbm, mid_hbm, utab_hbm, mtab_hbm, out_hbm,
                uid_v, mid_v, du_v, dm_v, out_v, sem,
                *, b_per_w, num_cores):
    wid = lax.axis_index("s") * num_cores + lax.axis_index("c")
    base = wid * b_per_w

    pltpu.sync_copy(uid_hbm.at[pl.ds(base, b_per_w)], uid_v)
    pltpu.sync_copy(mid_hbm.at[pl.ds(base, b_per_w)], mid_v)

    lane = lax.iota(jnp.int32, 16)

    for p in range(b_per_w // _HALF):
        p0 = p * _HALF

        def fire(g, _):
            k0 = g * 16
            bu = lax.shift_right_logical(uid_v[pl.ds(p0 + k0, 16)], 2)
            bm = lax.shift_right_logical(mid_v[pl.ds(p0 + k0, 16)], 2)
            for k in range(16):
                pltpu.async_copy(utab_hbm.at[pl.ds(bu[k], 1), :],
                                 du_v.at[pl.ds(k0 + k, 1), :], sem)
                pltpu.async_copy(mtab_hbm.at[pl.ds(bm[k], 1), :],
                                 dm_v.at[pl.ds(k0 + k, 1), :], sem)
            return 0

        lax.fori_loop(0, _HALF // 16, fire, 0)

        pltpu.make_async_copy(utab_hbm.at[pl.ds(0, _HALF), :], du_v,
                              sem).wait()
        pltpu.make_async_copy(mtab_hbm.at[pl.ds(0, _HALF), :], dm_v,
                              sem).wait()

        def group(g, _):
            k0 = g * 16
            uv = uid_v[pl.ds(p0 + k0, 16)]
            mv = mid_v[pl.ds(p0 + k0, 16)]
            offu = (uv & 3) * _EMBED
            offm = (mv & 3) * _EMBED
            rows = k0 + lane
            acc = jnp.zeros((16,), jnp.float32)
            cu = lane & (_EMBED - 1)
            for _d in range(_EMBED):
                u = plsc.load_gather(du_v, [rows, offu + cu])
                m = plsc.load_gather(dm_v, [rows, offm + cu])
                acc = acc + u * m
                cu = (cu + 1) & (_EMBED - 1)
            out_v[pl.ds(p0 + k0, 16)] = acc
            return 0

        lax.fori_loop(0, _HALF // 16, group, 0)

    pltpu.sync_copy(out_v, out_hbm.at[pl.ds(base, b_per_w)])


def kernel(user_ids, movie_ids, user_table, movie_table):
    batch = user_ids.shape[0]
    info = plsc.get_sparse_core_info()
    nw = info.num_cores * info.num_subcores
    b_per_w = batch // nw
    mesh = plsc.VectorSubcoreMesh(core_axis_name="c", subcore_axis_name="s")

    ut = user_table.reshape(-1, 128)
    mt = movie_table.reshape(-1, 128)

    run = pl.kernel(
        functools.partial(_dot_kernel, b_per_w=b_per_w,
                          num_cores=info.num_cores),
        mesh=mesh,
        compiler_params=pltpu.CompilerParams(needs_layout_passes=False),
        out_type=jax.ShapeDtypeStruct((batch,), jnp.float32),
        scratch_types=[
            pltpu.VMEM((b_per_w,), jnp.int32),
            pltpu.VMEM((b_per_w,), jnp.int32),
            pltpu.VMEM((_HALF, 128), jnp.float32),
            pltpu.VMEM((_HALF, 128), jnp.float32),
            pltpu.VMEM((b_per_w,), jnp.float32),
            pltpu.SemaphoreType.DMA,
        ],
    )
    return run(user_ids.astype(jnp.int32), movie_ids.astype(jnp.int32),
               ut, mt)
